# Initial kernel scaffold; baseline (speedup 1.0000x reference)
import jax
import jax.numpy as jnp
from jax.experimental import pallas as pl


def kernel(embedding, memory_bank):
    raise NotImplementedError("write your pallas kernel here")


# speedup vs baseline: 1.3424x; 1.3424x over previous
import jax
import jax.numpy as jnp
from jax.experimental import pallas as pl
from jax.experimental.pallas import tpu as pltpu

B = 8
P = 784
W = 28
H = 28
D = 384
M = 16384
Q = B * P
KNN = 9

TQ = 448
TM = 1024
NQB = Q // TQ
NMB = M // TM

CH = 1024
NCH = M // CH


QBIAS = 65536
QSCALE = 64.0


def _ynorm_kernel(y_ref, yh_ref):
    y = y_ref[...]
    ones = jnp.ones((1, D), jnp.float32)
    yh_ref[...] = 32.0 * jax.lax.dot_general(
        ones, y * y, (((1,), (1,)), ((), ())), preferred_element_type=jnp.float32
    )


def _stage1_kernel(x_ref, y_ref, yh_ref, score_ref, loc_ref, best_ref):
    m = pl.program_id(1)

    @pl.when(m == 0)
    def _init():
        best_ref[...] = jnp.full((TQ, 1), jnp.iinfo(jnp.int32).max, jnp.int32)

    x = x_ref[...]
    y = y_ref[...]
    x64 = x * QSCALE
    dot = jax.lax.dot_general(
        x64, y, (((1,), (1,)), ((), ())), preferred_element_type=jnp.float32
    )
    yh = yh_ref[...]
    ci = (yh - dot).astype(jnp.int32)
    ii = jax.lax.broadcasted_iota(jnp.int32, (TQ, TM), 1)
    iadd = ii + (QBIAS * 16384 + m * TM)
    combined = ci * 16384 + iadd
    lmin = jnp.min(combined, axis=1, keepdims=True)
    best_ref[...] = jnp.minimum(best_ref[...], lmin)

    @pl.when(m == NMB - 1)
    def _fin():
        best = best_ref[...]
        vq = jax.lax.shift_right_arithmetic(best, 14) - QBIAS
        xnorm = jnp.sum(x * x, axis=1, keepdims=True)
        res = xnorm + vq.astype(jnp.float32) * (2.0 / QSCALE)
        score_ref[...] = jnp.sqrt(jnp.maximum(res, 1e-12))
        loc_ref[...] = jnp.bitwise_and(best, 16383)


def _stage2_kernel(ps_ref, loc_ref, emb_ref, bank_ref, out_ref):
    ps = ps_ref[...]
    loc = loc_ref[...]
    iP = jax.lax.broadcasted_iota(jnp.int32, (B, P), 1)
    mx = jnp.max(ps, axis=1, keepdims=True)
    mp = jnp.min(jnp.where(ps == mx, iP, P), axis=1, keepdims=True)
    score = mx
    nn_idx = jnp.sum(jnp.where(iP == mp, loc, 0), axis=1, keepdims=True)

    i_ch = jax.lax.broadcasted_iota(jnp.int32, (B, CH), 1)
    nn_acc = jnp.zeros((B, D), jnp.float32)
    for c in range(NCH):
        bt = bank_ref[pl.ds(c * CH, CH), :]
        oh = (i_ch + c * CH == nn_idx).astype(jnp.float32)
        nn_acc = nn_acc + jax.lax.dot_general(
            oh, bt, (((1,), (0,)), ((), ())), preferred_element_type=jnp.float32
        )

    i_e = jax.lax.broadcasted_iota(jnp.int32, (B, P), 1)
    i_b = jax.lax.broadcasted_iota(jnp.int32, (B, P), 0)
    mf_acc = jnp.zeros((B, D), jnp.float32)
    for c in range(B):
        et = emb_ref[pl.ds(c * P, P), :]
        ohq = ((i_e == mp) & (i_b == c)).astype(jnp.float32)
        mf_acc = mf_acc + jax.lax.dot_general(
            ohq, et, (((1,), (0,)), ((), ())), preferred_element_type=jnp.float32
        )

    z = jnp.concatenate([nn_acc, mf_acc], axis=0)
    zn = jnp.sum(z * z, axis=1, keepdims=True)
    res_chunks = []
    for c in range(NCH):
        bt = bank_ref[pl.ds(c * CH, CH), :]
        zz = jax.lax.dot_general(
            z, bt, (((1,), (1,)), ((), ())), preferred_element_type=jnp.float32
        )
        ynorm = jnp.sum(bt * bt, axis=1)
        res_chunks.append(zn - 2.0 * zz + ynorm[None, :])
    res = jnp.concatenate(res_chunks, axis=1)
    r2 = res[:B]
    rq = res[B:]

    iM = jax.lax.broadcasted_iota(jnp.int32, (B, M), 1)
    work = r2
    d3_cols = []
    for _ in range(KNN):
        mn = jnp.min(work, axis=1, keepdims=True)
        am = jnp.min(jnp.where(work == mn, iM, M), axis=1, keepdims=True)
        selk = iM == am
        raw = jnp.sum(jnp.where(selk, rq, 0.0), axis=1, keepdims=True)
        d3_cols.append(jnp.sqrt(jnp.maximum(raw, 1e-12)))
        work = jnp.where(selk, jnp.inf, work)
    d3 = jnp.concatenate(d3_cols, axis=1)

    dmx = jnp.max(d3, axis=1, keepdims=True)
    e = jnp.exp(d3 - dmx)
    s0 = e[:, 0:1] / jnp.sum(e, axis=1, keepdims=True)
    out_ref[...] = (1.0 - s0) * score


def kernel(embedding, memory_bank):
    yh = pl.pallas_call(
        _ynorm_kernel,
        grid=(NMB,),
        in_specs=[pl.BlockSpec((TM, D), lambda m: (m, 0))],
        out_specs=pl.BlockSpec((1, TM), lambda m: (0, m)),
        out_shape=jax.ShapeDtypeStruct((1, M), jnp.float32),
    )(memory_bank)

    score, loc = pl.pallas_call(
        _stage1_kernel,
        grid=(NQB, NMB),
        in_specs=[
            pl.BlockSpec((TQ, D), lambda q, m: (q, 0)),
            pl.BlockSpec((TM, D), lambda q, m: (m, 0)),
            pl.BlockSpec((1, TM), lambda q, m: (0, m)),
        ],
        out_specs=[
            pl.BlockSpec((TQ, 1), lambda q, m: (q, 0)),
            pl.BlockSpec((TQ, 1), lambda q, m: (q, 0)),
        ],
        out_shape=[
            jax.ShapeDtypeStruct((Q, 1), jnp.float32),
            jax.ShapeDtypeStruct((Q, 1), jnp.int32),
        ],
        scratch_shapes=[
            pltpu.VMEM((TQ, 1), jnp.int32),
        ],
        compiler_params=pltpu.CompilerParams(
            dimension_semantics=("arbitrary", "arbitrary"),
        ),
    )(embedding, memory_bank, yh)

    patch_scores = score.reshape(B, P)
    locations = loc.reshape(B, P)

    pred = pl.pallas_call(
        _stage2_kernel,
        out_shape=jax.ShapeDtypeStruct((B, 1), jnp.float32),
    )(patch_scores, locations, embedding, memory_bank)

    anomaly_map = patch_scores.reshape(B, 1, W, H)
    return anomaly_map, pred.reshape(B)


# speedup vs baseline: 2.0475x; 1.5252x over previous
import jax
import jax.numpy as jnp
from jax.experimental import pallas as pl
from jax.experimental.pallas import tpu as pltpu

B = 8
P = 784
W = 28
H = 28
D = 384
M = 16384
Q = B * P
KNN = 9

TQ = 448
TM = 2048
NQB = Q // TQ
NMB = M // TM

CH = 1024
NCH = M // CH


QBIAS = 65536
QSCALE = 64.0
QOFF = float(QBIAS) + 8388608.0


def _ynorm_kernel(y_ref, yh_ref):
    y = y_ref[...]
    ones = jnp.ones((1, D), jnp.float32)
    yh_ref[...] = 32.0 * jax.lax.dot_general(
        ones, y * y, (((1,), (1,)), ((), ())), preferred_element_type=jnp.float32
    ) + QOFF


def _stage1_kernel(x_ref, y_ref, yh_ref, score_ref, loc_ref, best_ref):
    m = pl.program_id(1)

    @pl.when(m == 0)
    def _init():
        best_ref[...] = jnp.full((TQ, 1), jnp.inf, jnp.float32)

    x = x_ref[...]
    y = y_ref[...]
    dot = jax.lax.dot_general(
        x, y, (((1,), (1,)), ((), ())), preferred_element_type=jnp.float32
    )
    yh = yh_ref[...]
    t = yh - dot
    bits = jax.lax.bitcast_convert_type(t, jnp.int32)
    ii = jax.lax.broadcasted_iota(jnp.int32, (1, TM), 1) + m * TM
    combined = jax.lax.shift_left(bits, 14) + ii
    cf = jax.lax.bitcast_convert_type(combined, jnp.float32)
    lmin = jnp.min(cf, axis=1, keepdims=True)
    best_ref[...] = jnp.minimum(best_ref[...], lmin)

    @pl.when(m == NMB - 1)
    def _fin():
        best = jax.lax.bitcast_convert_type(best_ref[...], jnp.int32)
        vq = jax.lax.shift_right_arithmetic(best, 14) - QBIAS
        xf = x.astype(jnp.float32)
        xnorm = jnp.sum(xf * xf, axis=1, keepdims=True) * (1.0 / (QSCALE * QSCALE))
        res = xnorm + vq.astype(jnp.float32) * (2.0 / QSCALE)
        score_ref[...] = jnp.sqrt(jnp.maximum(res, 1e-12))
        loc_ref[...] = jnp.bitwise_and(best, 16383)


def _stage2_kernel(ps_ref, loc_ref, emb_ref, bank_ref, out_ref):
    ps = ps_ref[...]
    loc = loc_ref[...]
    iP = jax.lax.broadcasted_iota(jnp.int32, (B, P), 1)
    mx = jnp.max(ps, axis=1, keepdims=True)
    mp = jnp.min(jnp.where(ps == mx, iP, P), axis=1, keepdims=True)
    score = mx
    nn_idx = jnp.sum(jnp.where(iP == mp, loc, 0), axis=1, keepdims=True)

    i_ch = jax.lax.broadcasted_iota(jnp.int32, (B, CH), 1)
    nn_acc = jnp.zeros((B, D), jnp.float32)
    for c in range(NCH):
        bt = bank_ref[pl.ds(c * CH, CH), :]
        oh = (i_ch + c * CH == nn_idx).astype(jnp.float32)
        nn_acc = nn_acc + jax.lax.dot_general(
            oh, bt, (((1,), (0,)), ((), ())), preferred_element_type=jnp.float32
        )

    i_e = jax.lax.broadcasted_iota(jnp.int32, (B, P), 1)
    i_b = jax.lax.broadcasted_iota(jnp.int32, (B, P), 0)
    mf_acc = jnp.zeros((B, D), jnp.float32)
    for c in range(B):
        et = emb_ref[pl.ds(c * P, P), :]
        ohq = ((i_e == mp) & (i_b == c)).astype(jnp.float32)
        mf_acc = mf_acc + jax.lax.dot_general(
            ohq, et, (((1,), (0,)), ((), ())), preferred_element_type=jnp.float32
        )

    z = jnp.concatenate([nn_acc, mf_acc], axis=0)
    zn = jnp.sum(z * z, axis=1, keepdims=True)
    res_chunks = []
    for c in range(NCH):
        bt = bank_ref[pl.ds(c * CH, CH), :]
        zz = jax.lax.dot_general(
            z, bt, (((1,), (1,)), ((), ())), preferred_element_type=jnp.float32
        )
        ynorm = jnp.sum(bt * bt, axis=1)
        res_chunks.append(zn - 2.0 * zz + ynorm[None, :])
    res = jnp.concatenate(res_chunks, axis=1)
    r2 = res[:B]
    rq = res[B:]

    iM = jax.lax.broadcasted_iota(jnp.int32, (B, M), 1)
    work = r2
    d3_cols = []
    for _ in range(KNN):
        mn = jnp.min(work, axis=1, keepdims=True)
        am = jnp.min(jnp.where(work == mn, iM, M), axis=1, keepdims=True)
        selk = iM == am
        raw = jnp.sum(jnp.where(selk, rq, 0.0), axis=1, keepdims=True)
        d3_cols.append(jnp.sqrt(jnp.maximum(raw, 1e-12)))
        work = jnp.where(selk, jnp.inf, work)
    d3 = jnp.concatenate(d3_cols, axis=1)

    dmx = jnp.max(d3, axis=1, keepdims=True)
    e = jnp.exp(d3 - dmx)
    s0 = e[:, 0:1] / jnp.sum(e, axis=1, keepdims=True)
    out_ref[...] = (1.0 - s0) * score


def kernel(embedding, memory_bank):
    yh = pl.pallas_call(
        _ynorm_kernel,
        grid=(NMB,),
        in_specs=[pl.BlockSpec((TM, D), lambda m: (m, 0))],
        out_specs=pl.BlockSpec((1, TM), lambda m: (0, m)),
        out_shape=jax.ShapeDtypeStruct((1, M), jnp.float32),
    )(memory_bank)

    score, loc = pl.pallas_call(
        _stage1_kernel,
        grid=(NQB, NMB),
        in_specs=[
            pl.BlockSpec((TQ, D), lambda q, m: (q, 0)),
            pl.BlockSpec((TM, D), lambda q, m: (m, 0)),
            pl.BlockSpec((1, TM), lambda q, m: (0, m)),
        ],
        out_specs=[
            pl.BlockSpec((TQ, 1), lambda q, m: (q, 0)),
            pl.BlockSpec((TQ, 1), lambda q, m: (q, 0)),
        ],
        out_shape=[
            jax.ShapeDtypeStruct((Q, 1), jnp.float32),
            jax.ShapeDtypeStruct((Q, 1), jnp.int32),
        ],
        scratch_shapes=[
            pltpu.VMEM((TQ, 1), jnp.float32),
        ],
        compiler_params=pltpu.CompilerParams(
            dimension_semantics=("arbitrary", "arbitrary"),
        ),
    )(
        (embedding * QSCALE).astype(jnp.bfloat16),
        memory_bank.astype(jnp.bfloat16),
        yh,
    )

    patch_scores = score.reshape(B, P)
    locations = loc.reshape(B, P)

    pred = pl.pallas_call(
        _stage2_kernel,
        out_shape=jax.ShapeDtypeStruct((B, 1), jnp.float32),
    )(patch_scores, locations, embedding, memory_bank)

    anomaly_map = patch_scores.reshape(B, 1, W, H)
    return anomaly_map, pred.reshape(B)


# speedup vs baseline: 2.4931x; 1.2176x over previous
import jax
import jax.numpy as jnp
from jax.experimental import pallas as pl
from jax.experimental.pallas import tpu as pltpu

B = 8
P = 784
W = 28
H = 28
D = 384
M = 16384
Q = B * P
KNN = 9

TQ = 896
TM = 2048
NQB = Q // TQ
NMB = M // TM

CH = 1024
NCH = M // CH


QBIAS = 65536
QSCALE = 64.0
QOFF = float(QBIAS) + 8388608.0


def _ynorm_kernel(y_ref, yh_ref, ybf_ref):
    y = y_ref[...]
    ones = jnp.ones((1, D), jnp.float32)
    yh_ref[...] = 32.0 * jax.lax.dot_general(
        ones, y * y, (((1,), (1,)), ((), ())), preferred_element_type=jnp.float32
    ) + QOFF
    ybf_ref[...] = y.astype(jnp.bfloat16)


def _stage1_kernel(x_ref, y_ref, yh_ref, score_ref, loc_ref, best_ref):
    m = pl.program_id(1)

    @pl.when(m == 0)
    def _init():
        best_ref[...] = jnp.full((TQ, 1), jnp.inf, jnp.float32)

    x = x_ref[...]
    y = y_ref[...]
    dot = jax.lax.dot_general(
        x, y, (((1,), (1,)), ((), ())), preferred_element_type=jnp.float32
    )
    yh = yh_ref[...]
    t = yh - dot
    bits = jax.lax.bitcast_convert_type(t, jnp.int32)
    ii = jax.lax.broadcasted_iota(jnp.int32, (1, TM), 1) + m * TM
    combined = jax.lax.shift_left(bits, 14) + ii
    cf = jax.lax.bitcast_convert_type(combined, jnp.float32)
    lmin = jnp.min(cf, axis=1, keepdims=True)
    best_ref[...] = jnp.minimum(best_ref[...], lmin)

    @pl.when(m == NMB - 1)
    def _fin():
        best = jax.lax.bitcast_convert_type(best_ref[...], jnp.int32)
        vq = jax.lax.shift_right_arithmetic(best, 14) - QBIAS
        xf = x.astype(jnp.float32)
        xnorm = jnp.sum(xf * xf, axis=1, keepdims=True) * (1.0 / (QSCALE * QSCALE))
        res = xnorm + vq.astype(jnp.float32) * (2.0 / QSCALE)
        score_ref[...] = jnp.sqrt(jnp.maximum(res, 1e-12))
        loc_ref[...] = jnp.bitwise_and(best, 16383)


def _stage2_kernel(ps_ref, loc_ref, emb_ref, bank_ref, yh_ref, out_ref):
    ps = ps_ref[...]
    loc = loc_ref[...]
    iP = jax.lax.broadcasted_iota(jnp.int32, (B, P), 1)
    mx = jnp.max(ps, axis=1, keepdims=True)
    mp = jnp.min(jnp.where(ps == mx, iP, P), axis=1, keepdims=True)
    score = mx
    nn_idx = jnp.sum(jnp.where(iP == mp, loc, 0), axis=1, keepdims=True)

    i_ch = jax.lax.broadcasted_iota(jnp.int32, (B, CH), 1)
    nn_acc = jnp.zeros((B, D), jnp.float32)
    for c in range(NCH):
        bt = bank_ref[pl.ds(c * CH, CH), :]
        oh = (i_ch + c * CH == nn_idx).astype(jnp.bfloat16)
        nn_acc = nn_acc + jax.lax.dot_general(
            oh, bt, (((1,), (0,)), ((), ())), preferred_element_type=jnp.float32
        )

    i_e = jax.lax.broadcasted_iota(jnp.int32, (B, P), 1)
    i_b = jax.lax.broadcasted_iota(jnp.int32, (B, P), 0)
    mf_acc = jnp.zeros((B, D), jnp.float32)
    for c in range(B):
        et = emb_ref[pl.ds(c * P, P), :]
        ohq = ((i_e == mp) & (i_b == c)).astype(jnp.bfloat16)
        mf_acc = mf_acc + jax.lax.dot_general(
            ohq, et, (((1,), (0,)), ((), ())), preferred_element_type=jnp.float32
        )

    z = jnp.concatenate([nn_acc, mf_acc * (1.0 / QSCALE)], axis=0)
    zbf = z.astype(jnp.bfloat16)
    zn = jnp.sum(z * z, axis=1, keepdims=True)
    res_chunks = []
    for c in range(NCH):
        bt = bank_ref[pl.ds(c * CH, CH), :]
        zz = jax.lax.dot_general(
            zbf, bt, (((1,), (1,)), ((), ())), preferred_element_type=jnp.float32
        )
        ynorm = (yh_ref[:, pl.ds(c * CH, CH)] - QOFF) * (1.0 / 32.0)
        res_chunks.append(zn - 2.0 * zz + ynorm)
    res = jnp.concatenate(res_chunks, axis=1)
    r2 = res[:B]
    rq = res[B:]

    iM = jax.lax.broadcasted_iota(jnp.int32, (B, M), 1)
    work = r2
    d3_cols = []
    for _ in range(KNN):
        mn = jnp.min(work, axis=1, keepdims=True)
        am = jnp.min(jnp.where(work == mn, iM, M), axis=1, keepdims=True)
        selk = iM == am
        raw = jnp.sum(jnp.where(selk, rq, 0.0), axis=1, keepdims=True)
        d3_cols.append(jnp.sqrt(jnp.maximum(raw, 1e-12)))
        work = jnp.where(selk, jnp.inf, work)
    d3 = jnp.concatenate(d3_cols, axis=1)

    dmx = jnp.max(d3, axis=1, keepdims=True)
    e = jnp.exp(d3 - dmx)
    s0 = e[:, 0:1] / jnp.sum(e, axis=1, keepdims=True)
    out_ref[...] = (1.0 - s0) * score


def kernel(embedding, memory_bank):
    yh, ybf = pl.pallas_call(
        _ynorm_kernel,
        grid=(NMB,),
        in_specs=[pl.BlockSpec((TM, D), lambda m: (m, 0))],
        out_specs=[
            pl.BlockSpec((1, TM), lambda m: (0, m)),
            pl.BlockSpec((TM, D), lambda m: (m, 0)),
        ],
        out_shape=[
            jax.ShapeDtypeStruct((1, M), jnp.float32),
            jax.ShapeDtypeStruct((M, D), jnp.bfloat16),
        ],
    )(memory_bank)

    emb_bf = (embedding * QSCALE).astype(jnp.bfloat16)

    score, loc = pl.pallas_call(
        _stage1_kernel,
        grid=(NQB, NMB),
        in_specs=[
            pl.BlockSpec((TQ, D), lambda q, m: (q, 0)),
            pl.BlockSpec((TM, D), lambda q, m: (m, 0)),
            pl.BlockSpec((1, TM), lambda q, m: (0, m)),
        ],
        out_specs=[
            pl.BlockSpec((TQ, 1), lambda q, m: (q, 0)),
            pl.BlockSpec((TQ, 1), lambda q, m: (q, 0)),
        ],
        out_shape=[
            jax.ShapeDtypeStruct((Q, 1), jnp.float32),
            jax.ShapeDtypeStruct((Q, 1), jnp.int32),
        ],
        scratch_shapes=[
            pltpu.VMEM((TQ, 1), jnp.float32),
        ],
        compiler_params=pltpu.CompilerParams(
            dimension_semantics=("arbitrary", "arbitrary"),
        ),
    )(emb_bf, ybf, yh)

    patch_scores = score.reshape(B, P)
    locations = loc.reshape(B, P)

    pred = pl.pallas_call(
        _stage2_kernel,
        out_shape=jax.ShapeDtypeStruct((B, 1), jnp.float32),
    )(patch_scores, locations, emb_bf, ybf, yh)

    anomaly_map = patch_scores.reshape(B, 1, W, H)
    return anomaly_map, pred.reshape(B)


# speedup vs baseline: 2.7489x; 1.1026x over previous
import jax
import jax.numpy as jnp
from jax.experimental import pallas as pl
from jax.experimental.pallas import tpu as pltpu

B = 8
P = 784
W = 28
H = 28
D = 384
M = 16384
Q = B * P
KNN = 9

TQ = 896
TM = 8192
NQB = Q // TQ
NMB = M // TM

CH = 1024
NCH = M // CH


QBIAS = 65536
QSCALE = 64.0
QOFF = float(QBIAS) + 8388608.0


def _ynorm_kernel(y_ref, yh_ref, ybf_ref):
    y = y_ref[...]
    ones = jnp.ones((1, D), jnp.float32)
    yh_ref[...] = 32.0 * jax.lax.dot_general(
        ones, y * y, (((1,), (1,)), ((), ())), preferred_element_type=jnp.float32
    ) + QOFF
    ybf_ref[...] = y.astype(jnp.bfloat16)


def _stage1_kernel(x_ref, y_ref, yh_ref, score_ref, loc_ref, best_ref):
    m = pl.program_id(1)

    @pl.when(m == 0)
    def _init():
        best_ref[...] = jnp.full((TQ, 1), jnp.inf, jnp.float32)

    x = x_ref[...]
    y = y_ref[...]
    dot = jax.lax.dot_general(
        x, y, (((1,), (1,)), ((), ())), preferred_element_type=jnp.float32
    )
    yh = yh_ref[...]
    t = yh - dot
    bits = jax.lax.bitcast_convert_type(t, jnp.int32)
    ii = jax.lax.broadcasted_iota(jnp.int32, (1, TM), 1) + m * TM
    combined = jax.lax.shift_left(bits, 14) + ii
    cf = jax.lax.bitcast_convert_type(combined, jnp.float32)
    lmin = jnp.min(cf, axis=1, keepdims=True)
    best_ref[...] = jnp.minimum(best_ref[...], lmin)

    @pl.when(m == NMB - 1)
    def _fin():
        best = jax.lax.bitcast_convert_type(best_ref[...], jnp.int32)
        vq = jax.lax.shift_right_arithmetic(best, 14) - QBIAS
        xf = x.astype(jnp.float32)
        xnorm = jnp.sum(xf * xf, axis=1, keepdims=True) * (1.0 / (QSCALE * QSCALE))
        res = xnorm + vq.astype(jnp.float32) * (2.0 / QSCALE)
        score_ref[...] = jnp.sqrt(jnp.maximum(res, 1e-12))
        loc_ref[...] = jnp.bitwise_and(best, 16383)


def _stage2_kernel(ps_ref, loc_ref, emb_ref, bank_ref, yh_ref, out_ref):
    ps = ps_ref[...]
    loc = loc_ref[...]
    iP = jax.lax.broadcasted_iota(jnp.int32, (B, P), 1)
    mx = jnp.max(ps, axis=1, keepdims=True)
    mp = jnp.min(jnp.where(ps == mx, iP, P), axis=1, keepdims=True)
    score = mx
    nn_idx = jnp.sum(jnp.where(iP == mp, loc, 0), axis=1, keepdims=True)

    i_ch = jax.lax.broadcasted_iota(jnp.int32, (B, CH), 1)
    nn_acc = jnp.zeros((B, D), jnp.float32)
    for c in range(NCH):
        bt = bank_ref[pl.ds(c * CH, CH), :]
        oh = (i_ch + c * CH == nn_idx).astype(jnp.bfloat16)
        nn_acc = nn_acc + jax.lax.dot_general(
            oh, bt, (((1,), (0,)), ((), ())), preferred_element_type=jnp.float32
        )

    i_e = jax.lax.broadcasted_iota(jnp.int32, (B, P), 1)
    i_b = jax.lax.broadcasted_iota(jnp.int32, (B, P), 0)
    mf_acc = jnp.zeros((B, D), jnp.float32)
    for c in range(B):
        et = emb_ref[pl.ds(c * P, P), :]
        ohq = ((i_e == mp) & (i_b == c)).astype(jnp.bfloat16)
        mf_acc = mf_acc + jax.lax.dot_general(
            ohq, et, (((1,), (0,)), ((), ())), preferred_element_type=jnp.float32
        )

    z = jnp.concatenate([nn_acc, mf_acc * (1.0 / QSCALE)], axis=0)
    zbf = z.astype(jnp.bfloat16)
    zn = jnp.sum(z * z, axis=1, keepdims=True)
    res_chunks = []
    for c in range(NCH):
        bt = bank_ref[pl.ds(c * CH, CH), :]
        zz = jax.lax.dot_general(
            zbf, bt, (((1,), (1,)), ((), ())), preferred_element_type=jnp.float32
        )
        ynorm = (yh_ref[:, pl.ds(c * CH, CH)] - QOFF) * (1.0 / 32.0)
        res_chunks.append(zn - 2.0 * zz + ynorm)
    res = jnp.concatenate(res_chunks, axis=1)
    r2 = res[:B]
    rq = res[B:]

    iM = jax.lax.broadcasted_iota(jnp.int32, (B, M), 1)
    work = r2
    d3_cols = []
    for _ in range(KNN):
        mn = jnp.min(work, axis=1, keepdims=True)
        am = jnp.min(jnp.where(work == mn, iM, M), axis=1, keepdims=True)
        selk = iM == am
        raw = jnp.sum(jnp.where(selk, rq, 0.0), axis=1, keepdims=True)
        d3_cols.append(jnp.sqrt(jnp.maximum(raw, 1e-12)))
        work = jnp.where(selk, jnp.inf, work)
    d3 = jnp.concatenate(d3_cols, axis=1)

    dmx = jnp.max(d3, axis=1, keepdims=True)
    e = jnp.exp(d3 - dmx)
    s0 = e[:, 0:1] / jnp.sum(e, axis=1, keepdims=True)
    out_ref[...] = (1.0 - s0) * score


def kernel(embedding, memory_bank):
    yh, ybf = pl.pallas_call(
        _ynorm_kernel,
        grid=(NMB,),
        in_specs=[pl.BlockSpec((TM, D), lambda m: (m, 0))],
        out_specs=[
            pl.BlockSpec((1, TM), lambda m: (0, m)),
            pl.BlockSpec((TM, D), lambda m: (m, 0)),
        ],
        out_shape=[
            jax.ShapeDtypeStruct((1, M), jnp.float32),
            jax.ShapeDtypeStruct((M, D), jnp.bfloat16),
        ],
    )(memory_bank)

    emb_bf = (embedding * QSCALE).astype(jnp.bfloat16)

    score, loc = pl.pallas_call(
        _stage1_kernel,
        grid=(NQB, NMB),
        in_specs=[
            pl.BlockSpec((TQ, D), lambda q, m: (q, 0)),
            pl.BlockSpec((TM, D), lambda q, m: (m, 0)),
            pl.BlockSpec((1, TM), lambda q, m: (0, m)),
        ],
        out_specs=[
            pl.BlockSpec((TQ, 1), lambda q, m: (q, 0)),
            pl.BlockSpec((TQ, 1), lambda q, m: (q, 0)),
        ],
        out_shape=[
            jax.ShapeDtypeStruct((Q, 1), jnp.float32),
            jax.ShapeDtypeStruct((Q, 1), jnp.int32),
        ],
        scratch_shapes=[
            pltpu.VMEM((TQ, 1), jnp.float32),
        ],
        compiler_params=pltpu.CompilerParams(
            dimension_semantics=("arbitrary", "arbitrary"),
        ),
    )(emb_bf, ybf, yh)

    patch_scores = score.reshape(B, P)
    locations = loc.reshape(B, P)

    pred = pl.pallas_call(
        _stage2_kernel,
        out_shape=jax.ShapeDtypeStruct((B, 1), jnp.float32),
    )(patch_scores, locations, emb_bf, ybf, yh)

    anomaly_map = patch_scores.reshape(B, 1, W, H)
    return anomaly_map, pred.reshape(B)
